# Initial kernel scaffold; baseline (speedup 1.0000x reference)
import jax
import jax.numpy as jnp
from jax.experimental import pallas as pl


def kernel(positions, directions, hash_table, base_W1, base_W2, head_W1, head_W2, head_W3):
    raise NotImplementedError("write your pallas kernel here")


# speedup vs baseline: 1.4201x; 1.4201x over previous
import functools

import numpy as np
import jax
import jax.numpy as jnp
from jax import lax
from jax.experimental import pallas as pl
from jax.experimental.pallas import tpu as pltpu
from jax.experimental.pallas import tpu_sc as plsc

N = 262144
N_LEVELS = 16
LOG2_T = 19
T = 1 << LOG2_T
MASK = T - 1
SCALE = float(np.exp((np.log(4096.0) - np.log(16.0)) / (N_LEVELS - 1)))
RES = [float(np.floor(16.0 * SCALE ** l)) for l in range(N_LEVELS)]
P1I = int(np.uint32(2654435761).astype(np.int32))
P2I = int(np.uint32(805459861).astype(np.int32))

NW = 32
SPT = N // NW
B = 128
NG = B // 16
NCHUNK = SPT // B


def _sc_encode_body(plane0_hbm, plane1_hbm, posT_hbm, out_hbm, xn_v,
                    i0, i1, i2, i3, i4, i5, i6, i7,
                    w_v, f0_v, f1_v, enc_v, sem):
    idx_refs = (i0, i1, i2, i3, i4, i5, i6, i7)
    wid = lax.axis_index("s") * 2 + lax.axis_index("c")
    base0 = wid * SPT

    def chunk_body(ci, carry):
        cbase = base0 + ci * B
        pltpu.sync_copy(posT_hbm.at[:, pl.ds(cbase, B)], xn_v)

        def norm_body(g, c):
            for d in range(3):
                p = xn_v[d, pl.ds(g * 16, 16)]
                xn_v[d, pl.ds(g * 16, 16)] = (p + 1.0) * 0.5
            return c

        lax.fori_loop(0, NG, norm_body, 0)

        for l in range(N_LEVELS):
            res = RES[l]
            lbase = l * T

            def idx_body(g, c, res=res, lbase=lbase):
                pid = []
                fr = []
                for d in range(3):
                    posd = xn_v[d, pl.ds(g * 16, 16)] * res
                    pi = posd.astype(jnp.int32)
                    pid.append(pi)
                    fr.append(posd - pi.astype(jnp.float32))
                hx = pid[0]
                hy = pid[1] * P1I
                hz = pid[2] * P2I
                hx1 = hx + 1
                hy1 = hy + P1I
                hz1 = hz + P2I
                exy = [hx ^ hy, hx1 ^ hy, hx ^ hy1, hx1 ^ hy1]
                fx1, fy1, fz1 = fr
                fx0 = 1.0 - fx1
                fy0 = 1.0 - fy1
                fz0 = 1.0 - fz1
                wxy = [fx0 * fy0, fx1 * fy0, fx0 * fy1, fx1 * fy1]
                for cc in range(8):
                    oz = (cc >> 2) & 1
                    k = cc & 3
                    h = exy[k] ^ (hz1 if oz else hz)
                    idx_refs[cc][pl.ds(g * 16, 16)] = (h & MASK) + lbase
                    w_v[cc, pl.ds(g * 16, 16)] = wxy[k] * (fz1 if oz else fz0)
                return c

            lax.fori_loop(0, NG, idx_body, 0)

            for cc in range(8):
                pltpu.async_copy(
                    plane0_hbm.at[idx_refs[cc]],
                    f0_v.at[pl.ds(cc * B, B)],
                    sem,
                )
                pltpu.async_copy(
                    plane1_hbm.at[idx_refs[cc]],
                    f1_v.at[pl.ds(cc * B, B)],
                    sem,
                )
            pltpu.make_async_copy(plane0_hbm.at[pl.ds(0, 8 * B)], f0_v, sem).wait()
            pltpu.make_async_copy(plane1_hbm.at[pl.ds(0, 8 * B)], f1_v, sem).wait()

            def comb_body(g, c, l=l):
                acc0 = jnp.zeros((16,), jnp.float32)
                acc1 = jnp.zeros((16,), jnp.float32)
                for cc in range(8):
                    wv = w_v[cc, pl.ds(g * 16, 16)]
                    r0 = f0_v[pl.ds(cc * B + g * 16, 16)]
                    r1 = f1_v[pl.ds(cc * B + g * 16, 16)]
                    acc0 = acc0 + wv * r0
                    acc1 = acc1 + wv * r1
                enc_v[2 * l, pl.ds(g * 16, 16)] = acc0
                enc_v[2 * l + 1, pl.ds(g * 16, 16)] = acc1
                return c

            lax.fori_loop(0, NG, comb_body, 0)

        pltpu.sync_copy(enc_v, out_hbm.at[:, pl.ds(cbase, B)])
        return carry

    lax.fori_loop(0, NCHUNK, chunk_body, 0)


_SC_SCRATCH = [
    pltpu.VMEM((3, B), jnp.float32),
    *[pltpu.VMEM((B,), jnp.int32) for _ in range(8)],
    pltpu.VMEM((8, B), jnp.float32),
    pltpu.VMEM((8 * B,), jnp.float32),
    pltpu.VMEM((8 * B,), jnp.float32),
    pltpu.VMEM((2 * N_LEVELS, B), jnp.float32),
    pltpu.SemaphoreType.DMA,
]


def _make_sc_encode(interpret=False):
    return functools.partial(
        pl.kernel,
        out_type=jax.ShapeDtypeStruct((2 * N_LEVELS, N), jnp.float32),
        mesh=plsc.VectorSubcoreMesh(
            core_axis_name="c", subcore_axis_name="s", num_cores=2, num_subcores=16
        ),
        scratch_types=_SC_SCRATCH,
        interpret=interpret,
    )(_sc_encode_body)


_sc_encode = _make_sc_encode()


BN = 1024


def _mlp_body(encT, posT, dirT, w1t, w2t, hw1t, hw2t, hw3t, rgbT, denT):
    e = encT[...]
    h = jnp.maximum(jnp.dot(w1t[...], e, preferred_element_type=jnp.float32), 0.0)
    out = jnp.dot(w2t[...], h, preferred_element_type=jnp.float32)

    p = posT[...]
    x = (p + 1.0) / 2.0
    x0, x1, x2 = x[0:1, :], x[1:2, :], x[2:3, :]
    sel = (x0 > 0.0) & (x0 < 1.0) & (x1 > 0.0) & (x1 < 1.0) & (x2 > 0.0) & (x2 < 1.0)
    den = jnp.exp(out[0:1, :] - 1.0) * sel.astype(jnp.float32)

    d = dirT[...]
    dx, dy, dz = d[0:1, :], d[1:2, :], d[2:3, :]
    xx, yy, zz = dx * dx, dy * dy, dz * dz
    comps = [
        jnp.full_like(dx, 0.28209479177387814),
        -0.48860251190291987 * dy,
        0.48860251190291987 * dz,
        -0.48860251190291987 * dx,
        1.0925484305920792 * dx * dy,
        -1.0925484305920792 * dy * dz,
        0.94617469575755997 * zz - 0.31539156525251999,
        -1.0925484305920792 * dx * dz,
        0.54627421529603959 * (xx - yy),
        0.59004358992664352 * dy * (3.0 * xx - yy),
        2.8906114426405538 * dx * dy * dz,
        0.45704579946446572 * dy * (4.0 * zz - xx - yy),
        0.3731763325901154 * dz * (2.0 * zz - 3.0 * xx - 3.0 * yy),
        0.45704579946446572 * dx * (4.0 * zz - xx - yy),
        1.4453057213202769 * dz * (xx - yy),
        0.59004358992664352 * dx * (xx - 3.0 * yy),
    ]
    sh = jnp.concatenate(comps, axis=0)
    hh = jnp.concatenate([sh, out[1:16, :], jnp.zeros((1, BN), jnp.float32)], axis=0)
    h1 = jnp.maximum(jnp.dot(hw1t[...], hh, preferred_element_type=jnp.float32), 0.0)
    h2 = jnp.maximum(jnp.dot(hw2t[...], h1, preferred_element_type=jnp.float32), 0.0)
    z = jnp.dot(hw3t[...], h2, preferred_element_type=jnp.float32)
    rgbT[...] = 1.0 / (1.0 + jnp.exp(-z))
    denT[...] = den


def _mlp(encT, posT, dirT, w1t, w2t, hw1t, hw2t, hw3t):
    grid = (N // BN,)
    return pl.pallas_call(
        _mlp_body,
        grid=grid,
        in_specs=[
            pl.BlockSpec((2 * N_LEVELS, BN), lambda i: (0, i)),
            pl.BlockSpec((3, BN), lambda i: (0, i)),
            pl.BlockSpec((3, BN), lambda i: (0, i)),
            pl.BlockSpec((64, 32), lambda i: (0, 0)),
            pl.BlockSpec((16, 64), lambda i: (0, 0)),
            pl.BlockSpec((64, 32), lambda i: (0, 0)),
            pl.BlockSpec((64, 64), lambda i: (0, 0)),
            pl.BlockSpec((3, 64), lambda i: (0, 0)),
        ],
        out_specs=[
            pl.BlockSpec((3, BN), lambda i: (0, i)),
            pl.BlockSpec((1, BN), lambda i: (0, i)),
        ],
        out_shape=[
            jax.ShapeDtypeStruct((3, N), jnp.float32),
            jax.ShapeDtypeStruct((1, N), jnp.float32),
        ],
    )(encT, posT, dirT, w1t, w2t, hw1t, hw2t, hw3t)


def kernel(positions, directions, hash_table, base_W1, base_W2, head_W1, head_W2, head_W3):
    posT = positions.T
    dirT = directions.T
    plane0 = hash_table[:, :, 0].reshape(N_LEVELS * T)
    plane1 = hash_table[:, :, 1].reshape(N_LEVELS * T)
    encT = _sc_encode(plane0, plane1, posT)
    w1t = base_W1.T
    w2t = base_W2.T
    hw1t = jnp.pad(head_W1, ((0, 1), (0, 0))).T
    hw2t = head_W2.T
    hw3t = head_W3.T
    rgbT, denT = _mlp(encT, posT, dirT, w1t, w2t, hw1t, hw2t, hw3t)
    return rgbT.T, denT.T
